# Initial kernel scaffold; baseline (speedup 1.0000x reference)
import jax
import jax.numpy as jnp
from jax.experimental import pallas as pl


def kernel(x, edge_index, weight, bias, attention, lin_w, lin_b):
    raise NotImplementedError("write your pallas kernel here")


# speedup vs baseline: 25.5620x; 25.5620x over previous
import functools

import jax
import jax.numpy as jnp
from jax import lax
from jax.experimental import pallas as pl
from jax.experimental.pallas import tpu as pltpu
from jax.experimental.pallas import tpu_sc as plsc

_LANES = 16


def _make_sc_seg(n_nodes, n_rows):
    NC, NS = 2, 16
    p1_rows = n_rows // NS
    p2_rows = n_rows // (NC * NS)
    nvec = n_nodes // _LANES

    mesh = plsc.VectorSubcoreMesh(core_axis_name="c", subcore_axis_name="s")

    @functools.partial(
        pl.kernel,
        mesh=mesh,
        out_type=[
            jax.ShapeDtypeStruct((n_nodes,), jnp.float32),
            jax.ShapeDtypeStruct((NC, n_nodes), jnp.float32),
        ],
        scratch_types=[
            pltpu.VMEM((p1_rows, 128), jnp.int32),
            pltpu.VMEM((p2_rows, 128), jnp.int32),
            pltpu.VMEM((128,), jnp.float32),
            pltpu.VMEM((128,), jnp.float32),
            pltpu.VMEM((n_nodes,), jnp.float32),
            pltpu.VMEM_SHARED((n_nodes,), jnp.float32),
            pltpu.VMEM_SHARED((n_nodes,), jnp.float32),
            pltpu.VMEM_SHARED((n_nodes,), jnp.float32),
        ],
    )
    def sc_seg(src_hbm, dst_hbm, dis_out, s2_out,
               idx_v, dst_v, vals_v, ones_v, buf_v, sh_deg, sh_dis, sh_s):
        c = lax.axis_index("c")
        sid = lax.axis_index("s")

        for k in range(128 // _LANES):
            ones_v[pl.ds(k * _LANES, _LANES)] = jnp.ones((_LANES,), jnp.float32)

        def zero_body(i, carry):
            buf_v[pl.ds(i * _LANES, _LANES)] = jnp.zeros((_LANES,), jnp.float32)
            return carry
        lax.fori_loop(0, nvec, zero_body, 0)

        @pl.when(sid == 0)
        def _():
            pltpu.sync_copy(buf_v, sh_deg)
            pltpu.sync_copy(buf_v, sh_s)
        plsc.subcore_barrier()

        pltpu.sync_copy(src_hbm.at[pl.ds(sid * p1_rows, p1_rows)], idx_v)

        def p1_body(j, carry):
            pltpu.sync_copy(ones_v, sh_deg.at[idx_v.at[j]], add=True)
            return carry
        lax.fori_loop(0, p1_rows, p1_body, 0)
        plsc.subcore_barrier()

        @pl.when(sid == 0)
        def _():
            pltpu.sync_copy(sh_deg, buf_v)

            def dis_body(i, carry):
                d = buf_v[pl.ds(i * _LANES, _LANES)]
                bits = lax.bitcast_convert_type(d, jnp.int32)
                y = lax.bitcast_convert_type(
                    jnp.int32(0x5F3759DF) - (bits >> 1), jnp.float32)
                for _ in range(4):
                    y = y * (1.5 - 0.5 * d * y * y)
                y = jnp.where(d == 0.0, jnp.float32(jnp.inf), y)
                buf_v[pl.ds(i * _LANES, _LANES)] = y
                return carry
            lax.fori_loop(0, nvec, dis_body, 0)
            pltpu.sync_copy(buf_v, sh_dis)

        @pl.when(jnp.logical_and(c == 0, sid == 0))
        def _():
            pltpu.sync_copy(buf_v, dis_out)
        plsc.subcore_barrier()

        wid = sid * NC + c
        pltpu.sync_copy(src_hbm.at[pl.ds(wid * p2_rows, p2_rows)],
                        idx_v.at[pl.ds(0, p2_rows)])
        pltpu.sync_copy(dst_hbm.at[pl.ds(wid * p2_rows, p2_rows)], dst_v)

        def p2_body(j, carry):
            pltpu.sync_copy(sh_dis.at[dst_v.at[j]], vals_v)
            pltpu.sync_copy(vals_v, sh_s.at[idx_v.at[j]], add=True)
            return carry
        lax.fori_loop(0, p2_rows, p2_body, 0)
        plsc.subcore_barrier()

        @pl.when(sid == 0)
        def _():
            pltpu.sync_copy(sh_s, s2_out.at[c])

    return sc_seg


def _dense_body(x_ref, w_ref, att_ref, lw_ref, lb_ref, b_ref, dis_ref, s2_ref,
                o_ref):
    x = x_ref[...]
    dis = dis_ref[...]
    s = s2_ref[0:1, :] + s2_ref[1:2, :]
    f32 = jnp.float32
    u = jnp.dot(dis, x, preferred_element_type=f32,
                precision=lax.Precision.HIGHEST)
    v = jnp.dot(u, w_ref[...], preferred_element_type=f32,
                precision=lax.Precision.HIGHEST)
    a1 = att_ref[0:1, :]
    a2 = att_ref[1:2, :]
    w1 = lax.dot_general(a1, lw_ref[...], (((1,), (1,)), ((), ())),
                         preferred_element_type=f32,
                         precision=lax.Precision.HIGHEST)
    b1 = jnp.sum(lb_ref[...] * a1, keepdims=True)
    c2 = jnp.sum(v * a2, keepdims=True)
    pre = lax.dot_general(w1, x, (((1,), (1,)), ((), ())),
                          preferred_element_type=f32,
                          precision=lax.Precision.HIGHEST)
    pre = pre + b1 + s * c2
    alpha = jnp.where(pre >= 0, pre, 0.2 * pre)
    t = jnp.sum(v * alpha, keepdims=True)
    o_ref[...] = jnp.maximum(s * t + b_ref[...], 0.0)


def kernel(x, edge_index, weight, bias, attention, lin_w, lin_b):
    n, in_ch = x.shape
    out_ch = weight.shape[1]
    e = edge_index.shape[1]
    src2d = edge_index[0].reshape(e // 128, 128)
    dst2d = edge_index[1].reshape(e // 128, 128)

    dis, s2 = _make_sc_seg(n, e // 128)(src2d, dst2d)

    out = pl.pallas_call(
        _dense_body,
        out_shape=jax.ShapeDtypeStruct((1, n), jnp.float32),
    )(x, weight, attention.reshape(2, out_ch), lin_w,
      lin_b.reshape(1, out_ch), bias.reshape(1, out_ch),
      dis.reshape(1, n), s2)
    return out.reshape(n)


# speedup vs baseline: 30.3213x; 1.1862x over previous
import functools

import jax
import jax.numpy as jnp
from jax import lax
from jax.experimental import pallas as pl
from jax.experimental.pallas import tpu as pltpu
from jax.experimental.pallas import tpu_sc as plsc

_LANES = 16
_NS = 16


def _make_sc_seg(n_nodes, n_rows):
    rows_pt = n_rows // _NS
    nvec = n_nodes // _LANES
    P1C = 16
    P2C = 8

    mesh = plsc.VectorSubcoreMesh(core_axis_name="c", subcore_axis_name="s",
                                  num_cores=1)

    @functools.partial(
        pl.kernel,
        mesh=mesh,
        out_type=[
            jax.ShapeDtypeStruct((n_nodes,), jnp.float32),
            jax.ShapeDtypeStruct((n_nodes,), jnp.float32),
        ],
        scratch_types=[
            pltpu.VMEM((rows_pt, 128), jnp.int32),
            pltpu.VMEM((rows_pt, 128), jnp.int32),
            pltpu.VMEM((P2C, 128), jnp.float32),
            pltpu.VMEM((128,), jnp.float32),
            pltpu.VMEM((n_nodes,), jnp.float32),
            pltpu.VMEM_SHARED((n_nodes,), jnp.float32),
            pltpu.VMEM_SHARED((n_nodes,), jnp.float32),
            pltpu.VMEM_SHARED((n_nodes,), jnp.float32),
            pltpu.SemaphoreType.DMA,
            pltpu.SemaphoreType.DMA,
        ],
    )
    def sc_seg(src_hbm, dst_hbm, dis_out, s_out,
               idx_v, dst_v, vals_v, ones_v, buf_v, sh_deg, sh_dis, sh_s,
               sem_a, sem_b):
        sid = lax.axis_index("s")

        h_src = pltpu.async_copy(src_hbm.at[pl.ds(sid * rows_pt, rows_pt)],
                                 idx_v, sem_a)
        h_dst = pltpu.async_copy(dst_hbm.at[pl.ds(sid * rows_pt, rows_pt)],
                                 dst_v, sem_b)

        for k in range(128 // _LANES):
            ones_v[pl.ds(k * _LANES, _LANES)] = jnp.ones((_LANES,), jnp.float32)

        def zero_body(i, carry):
            buf_v[pl.ds(i * _LANES, _LANES)] = jnp.zeros((_LANES,), jnp.float32)
            return carry
        lax.fori_loop(0, nvec, zero_body, 0)

        @pl.when(sid == 0)
        def _():
            pltpu.sync_copy(buf_v, sh_deg)
            pltpu.sync_copy(buf_v, sh_s)
        plsc.subcore_barrier()
        h_src.wait()
        h_dst.wait()

        def p1_chunk(ci, carry):
            base = ci * P1C
            hs = [pltpu.async_copy(ones_v, sh_deg.at[idx_v.at[base + j]],
                                   sem_a, add=True)
                  for j in range(P1C)]
            for h in hs:
                h.wait()
            return carry
        lax.fori_loop(0, rows_pt // P1C, p1_chunk, 0)
        plsc.subcore_barrier()

        @pl.when(sid == 0)
        def _():
            pltpu.sync_copy(sh_deg, buf_v)

            def dis_body(i, carry):
                d = buf_v[pl.ds(i * _LANES, _LANES)]
                bits = lax.bitcast_convert_type(d, jnp.int32)
                y = lax.bitcast_convert_type(
                    jnp.int32(0x5F3759DF) - (bits >> 1), jnp.float32)
                for _ in range(4):
                    y = y * (1.5 - 0.5 * d * y * y)
                y = jnp.where(d == 0.0, jnp.float32(jnp.inf), y)
                buf_v[pl.ds(i * _LANES, _LANES)] = y
                return carry
            lax.fori_loop(0, nvec, dis_body, 0)
            pltpu.sync_copy(buf_v, sh_dis)
            pltpu.sync_copy(buf_v, dis_out)
        plsc.subcore_barrier()

        def p2_chunk(ci, carry):
            base = ci * P2C
            hg = [pltpu.async_copy(sh_dis.at[dst_v.at[base + j]],
                                   vals_v.at[j], sem_a)
                  for j in range(P2C)]
            for h in hg:
                h.wait()
            hs = [pltpu.async_copy(vals_v.at[j], sh_s.at[idx_v.at[base + j]],
                                   sem_b, add=True)
                  for j in range(P2C)]
            for h in hs:
                h.wait()
            return carry
        lax.fori_loop(0, rows_pt // P2C, p2_chunk, 0)
        plsc.subcore_barrier()

        @pl.when(sid == 0)
        def _():
            pltpu.sync_copy(sh_s, s_out)

    return sc_seg


def _dense_body(x_ref, w_ref, att_ref, lw_ref, lb_ref, b_ref, dis_ref, s_ref,
                o_ref):
    x = x_ref[...]
    dis = dis_ref[...]
    s = s_ref[...]
    f32 = jnp.float32
    hi = lax.Precision.HIGHEST
    u = jnp.dot(dis, x, preferred_element_type=f32, precision=hi)
    v = jnp.dot(u, w_ref[...], preferred_element_type=f32, precision=hi)
    a1 = att_ref[0:1, :]
    a2 = att_ref[1:2, :]
    w1 = lax.dot_general(a1, lw_ref[...], (((1,), (1,)), ((), ())),
                         preferred_element_type=f32, precision=hi)
    b1 = jnp.sum(lb_ref[...] * a1, keepdims=True)
    c2 = jnp.sum(v * a2, keepdims=True)
    pre = lax.dot_general(w1, x, (((1,), (1,)), ((), ())),
                          preferred_element_type=f32, precision=hi)
    pre = pre + b1 + s * c2
    alpha = jnp.where(pre >= 0, pre, 0.2 * pre)
    t = jnp.sum(v * alpha, keepdims=True)
    o_ref[...] = jnp.maximum(s * t + b_ref[...], 0.0)


def kernel(x, edge_index, weight, bias, attention, lin_w, lin_b):
    n, in_ch = x.shape
    out_ch = weight.shape[1]
    e = edge_index.shape[1]
    src2d = edge_index[0].reshape(e // 128, 128)
    dst2d = edge_index[1].reshape(e // 128, 128)

    dis, s = _make_sc_seg(n, e // 128)(src2d, dst2d)

    out = pl.pallas_call(
        _dense_body,
        out_shape=jax.ShapeDtypeStruct((1, n), jnp.float32),
    )(x, weight, attention.reshape(2, out_ch), lin_w,
      lin_b.reshape(1, out_ch), bias.reshape(1, out_ch),
      dis.reshape(1, n), s.reshape(1, n))
    return out.reshape(n)


# speedup vs baseline: 31.6022x; 1.0422x over previous
import functools

import jax
import jax.numpy as jnp
from jax import lax
from jax.experimental import pallas as pl
from jax.experimental.pallas import tpu as pltpu
from jax.experimental.pallas import tpu_sc as plsc

_LANES = 16
_NS = 16


def _make_sc_seg(n_nodes, n_rows):
    NC = 2
    rows_pt = n_rows // _NS
    rows_pw = n_rows // (NC * _NS)
    nvec = n_nodes // _LANES
    P1C = 16
    P2C = 8

    mesh = plsc.VectorSubcoreMesh(core_axis_name="c", subcore_axis_name="s",
                                  num_cores=NC)

    @functools.partial(
        pl.kernel,
        mesh=mesh,
        out_type=[
            jax.ShapeDtypeStruct((n_nodes,), jnp.float32),
            jax.ShapeDtypeStruct((NC, n_nodes), jnp.float32),
        ],
        scratch_types=[
            pltpu.VMEM((rows_pt, 128), jnp.int32),
            pltpu.VMEM((rows_pw, 128), jnp.int32),
            pltpu.VMEM((P2C, 128), jnp.float32),
            pltpu.VMEM((128,), jnp.float32),
            pltpu.VMEM((n_nodes,), jnp.float32),
            pltpu.VMEM_SHARED((n_nodes,), jnp.float32),
            pltpu.VMEM_SHARED((n_nodes,), jnp.float32),
            pltpu.VMEM_SHARED((n_nodes,), jnp.float32),
            pltpu.SemaphoreType.DMA,
            pltpu.SemaphoreType.DMA,
        ],
    )
    def sc_seg(src_hbm, dst_hbm, dis_out, s2_out,
               idx_v, dst_v, vals_v, ones_v, buf_v, sh_deg, sh_dis, sh_s,
               sem_a, sem_b):
        c = lax.axis_index("c")
        sid = lax.axis_index("s")
        wid = sid * NC + c

        h_src = pltpu.async_copy(src_hbm.at[pl.ds(sid * rows_pt, rows_pt)],
                                 idx_v, sem_a)
        h_dst = pltpu.async_copy(dst_hbm.at[pl.ds(wid * rows_pw, rows_pw)],
                                 dst_v, sem_b)

        for k in range(128 // _LANES):
            ones_v[pl.ds(k * _LANES, _LANES)] = jnp.ones((_LANES,), jnp.float32)

        def zero_body(i, carry):
            buf_v[pl.ds(i * _LANES, _LANES)] = jnp.zeros((_LANES,), jnp.float32)
            return carry
        lax.fori_loop(0, nvec, zero_body, 0)

        @pl.when(sid == 0)
        def _():
            pltpu.sync_copy(buf_v, sh_deg)
            pltpu.sync_copy(buf_v, sh_s)
        plsc.subcore_barrier()
        h_src.wait()
        h_dst.wait()

        def p1_chunk(ci, carry):
            base = ci * P1C
            hs = [pltpu.async_copy(ones_v, sh_deg.at[idx_v.at[base + j]],
                                   sem_a, add=True)
                  for j in range(P1C)]
            for h in hs:
                h.wait()
            return carry
        lax.fori_loop(0, rows_pt // P1C, p1_chunk, 0)
        plsc.subcore_barrier()

        @pl.when(sid == 0)
        def _():
            pltpu.sync_copy(sh_deg, buf_v)

            def dis_body(i, carry):
                d = buf_v[pl.ds(i * _LANES, _LANES)]
                bits = lax.bitcast_convert_type(d, jnp.int32)
                y = lax.bitcast_convert_type(
                    jnp.int32(0x5F3759DF) - (bits >> 1), jnp.float32)
                for _ in range(4):
                    y = y * (1.5 - 0.5 * d * y * y)
                y = jnp.where(d == 0.0, jnp.float32(jnp.inf), y)
                buf_v[pl.ds(i * _LANES, _LANES)] = y
                return carry
            lax.fori_loop(0, nvec, dis_body, 0)
            pltpu.sync_copy(buf_v, sh_dis)

        @pl.when(jnp.logical_and(c == 0, sid == 0))
        def _():
            pltpu.sync_copy(buf_v, dis_out)
        plsc.subcore_barrier()

        def p2_chunk(ci, carry):
            base = ci * P2C
            hg = [pltpu.async_copy(sh_dis.at[dst_v.at[base + j]],
                                   vals_v.at[j], sem_a)
                  for j in range(P2C)]
            for h in hg:
                h.wait()
            hs = [pltpu.async_copy(vals_v.at[j],
                                   sh_s.at[idx_v.at[c * rows_pw + base + j]],
                                   sem_b, add=True)
                  for j in range(P2C)]
            for h in hs:
                h.wait()
            return carry
        lax.fori_loop(0, rows_pw // P2C, p2_chunk, 0)
        plsc.subcore_barrier()

        @pl.when(sid == 0)
        def _():
            pltpu.sync_copy(sh_s, s2_out.at[c])

    return sc_seg


def _dense_body(x_ref, w_ref, att_ref, lw_ref, lb_ref, b_ref, dis_ref, s2_ref,
                o_ref):
    x = x_ref[...]
    dis = dis_ref[...]
    s = s2_ref[0:1, :] + s2_ref[1:2, :]
    f32 = jnp.float32
    hi = lax.Precision.HIGHEST
    u = jnp.dot(dis, x, preferred_element_type=f32, precision=hi)
    v = jnp.dot(u, w_ref[...], preferred_element_type=f32, precision=hi)
    a1 = att_ref[0:1, :]
    a2 = att_ref[1:2, :]
    w1 = lax.dot_general(a1, lw_ref[...], (((1,), (1,)), ((), ())),
                         preferred_element_type=f32, precision=hi)
    b1 = jnp.sum(lb_ref[...] * a1, keepdims=True)
    c2 = jnp.sum(v * a2, keepdims=True)
    pre = lax.dot_general(w1, x, (((1,), (1,)), ((), ())),
                          preferred_element_type=f32, precision=hi)
    pre = pre + b1 + s * c2
    alpha = jnp.where(pre >= 0, pre, 0.2 * pre)
    t = jnp.sum(v * alpha, keepdims=True)
    o_ref[...] = jnp.maximum(s * t + b_ref[...], 0.0)


def kernel(x, edge_index, weight, bias, attention, lin_w, lin_b):
    n, in_ch = x.shape
    out_ch = weight.shape[1]
    e = edge_index.shape[1]
    src2d = edge_index[0].reshape(e // 128, 128)
    dst2d = edge_index[1].reshape(e // 128, 128)

    dis, s2 = _make_sc_seg(n, e // 128)(src2d, dst2d)

    out = pl.pallas_call(
        _dense_body,
        out_shape=jax.ShapeDtypeStruct((1, n), jnp.float32),
    )(x, weight, attention.reshape(2, out_ch), lin_w,
      lin_b.reshape(1, out_ch), bias.reshape(1, out_ch),
      dis.reshape(1, n), s2)
    return out.reshape(n)


# speedup vs baseline: 36.4953x; 1.1548x over previous
import functools

import jax
import jax.numpy as jnp
from jax import lax
from jax.experimental import pallas as pl
from jax.experimental.pallas import tpu as pltpu
from jax.experimental.pallas import tpu_sc as plsc

_LANES = 16
_NS = 16


def _make_sc_seg(n_nodes, n_rows, in_ch):
    NC = 2
    NW = NC * _NS
    rows_pt = n_rows // _NS
    rows_pw = n_rows // NW
    blk = n_nodes // _NS
    xrows = n_nodes // NW
    nvec_blk = blk // _LANES
    P1C = 16
    P2C = 8

    mesh = plsc.VectorSubcoreMesh(core_axis_name="c", subcore_axis_name="s",
                                  num_cores=NC)

    @functools.partial(
        pl.kernel,
        mesh=mesh,
        out_type=[
            jax.ShapeDtypeStruct((NC, n_nodes), jnp.float32),
            jax.ShapeDtypeStruct((NC, in_ch), jnp.float32),
        ],
        scratch_types=[
            pltpu.VMEM((rows_pt, 128), jnp.int32),
            pltpu.VMEM((rows_pw, 128), jnp.int32),
            pltpu.VMEM((P2C, 128), jnp.float32),
            pltpu.VMEM((128,), jnp.float32),
            pltpu.VMEM((n_nodes,), jnp.float32),
            pltpu.VMEM((blk,), jnp.float32),
            pltpu.VMEM((xrows, in_ch), jnp.float32),
            pltpu.VMEM((xrows,), jnp.float32),
            pltpu.VMEM((in_ch,), jnp.float32),
            pltpu.VMEM((in_ch,), jnp.int32),
            pltpu.VMEM_SHARED((n_nodes,), jnp.float32),
            pltpu.VMEM_SHARED((n_nodes,), jnp.float32),
            pltpu.VMEM_SHARED((n_nodes,), jnp.float32),
            pltpu.VMEM_SHARED((in_ch,), jnp.float32),
            pltpu.SemaphoreType.DMA,
            pltpu.SemaphoreType.DMA,
            pltpu.SemaphoreType.DMA,
        ],
    )
    def sc_seg(src_hbm, dst_hbm, x_hbm, s2_out, u2_out,
               idx_v, dst_v, vals_v, ones_v, buf_v, dis_t, x_v, disx_v, u_v,
               iota_v, sh_deg, sh_dis, sh_s, sh_u, sem_a, sem_b, sem_c):
        c = lax.axis_index("c")
        sid = lax.axis_index("s")
        wid = sid * NC + c

        h_src = pltpu.async_copy(src_hbm.at[pl.ds(sid * rows_pt, rows_pt)],
                                 idx_v, sem_a)
        h_dst = pltpu.async_copy(dst_hbm.at[pl.ds(wid * rows_pw, rows_pw)],
                                 dst_v, sem_b)
        h_x = pltpu.async_copy(x_hbm.at[pl.ds(wid * xrows, xrows)], x_v, sem_c)

        for k in range(128 // _LANES):
            ones_v[pl.ds(k * _LANES, _LANES)] = jnp.ones((_LANES,), jnp.float32)
        for k in range(in_ch // _LANES):
            iota_v[pl.ds(k * _LANES, _LANES)] = (
                lax.iota(jnp.int32, _LANES) + (k * _LANES))

        def zero_body(i, carry):
            buf_v[pl.ds(i * _LANES, _LANES)] = jnp.zeros((_LANES,), jnp.float32)
            return carry
        lax.fori_loop(0, n_nodes // _LANES, zero_body, 0)
        for k in range(in_ch // _LANES):
            u_v[pl.ds(k * _LANES, _LANES)] = jnp.zeros((_LANES,), jnp.float32)

        @pl.when(sid == 0)
        def _():
            pltpu.sync_copy(buf_v, sh_deg)
            pltpu.sync_copy(buf_v, sh_s)
            pltpu.sync_copy(u_v, sh_u)
        plsc.subcore_barrier()
        h_src.wait()

        def p1_chunk(ci, carry):
            base = ci * P1C
            hs = [pltpu.async_copy(ones_v, sh_deg.at[idx_v.at[base + j]],
                                   sem_a, add=True)
                  for j in range(P1C)]
            for h in hs:
                h.wait()
            return carry
        lax.fori_loop(0, rows_pt // P1C, p1_chunk, 0)
        plsc.subcore_barrier()

        pltpu.sync_copy(sh_deg.at[pl.ds(sid * blk, blk)], dis_t)

        def dis_body(i, carry):
            d = dis_t[pl.ds(i * _LANES, _LANES)]
            bits = lax.bitcast_convert_type(d, jnp.int32)
            y = lax.bitcast_convert_type(
                jnp.int32(0x5F3759DF) - (bits >> 1), jnp.float32)
            for _ in range(4):
                y = y * (1.5 - 0.5 * d * y * y)
            y = jnp.where(d == 0.0, jnp.float32(jnp.inf), y)
            dis_t[pl.ds(i * _LANES, _LANES)] = y
            return carry
        lax.fori_loop(0, nvec_blk, dis_body, 0)
        pltpu.sync_copy(dis_t, sh_dis.at[pl.ds(sid * blk, blk)])
        plsc.subcore_barrier()

        def p2_chunk(ci, carry):
            base = ci * P2C
            hg = [pltpu.async_copy(sh_dis.at[dst_v.at[base + j]],
                                   vals_v.at[j], sem_a)
                  for j in range(P2C)]
            for h in hg:
                h.wait()
            hs = [pltpu.async_copy(vals_v.at[j],
                                   sh_s.at[idx_v.at[c * rows_pw + base + j]],
                                   sem_b, add=True)
                  for j in range(P2C)]
            for h in hs:
                h.wait()
            return carry
        lax.fori_loop(0, rows_pw // P2C, p2_chunk, 0)

        pltpu.sync_copy(sh_dis.at[pl.ds(wid * xrows, xrows)], disx_v)
        h_x.wait()

        def u_body(g, carry):
            dvec = disx_v[pl.ds(g * _LANES, _LANES)]
            for l in range(_LANES):
                d = jnp.full((_LANES,), dvec[l], jnp.float32)
                for k in range(in_ch // _LANES):
                    sl = pl.ds(k * _LANES, _LANES)
                    u_v[sl] = u_v[sl] + d * x_v[g * _LANES + l, sl]
            return carry
        lax.fori_loop(0, xrows // _LANES, u_body, 0)
        pltpu.sync_copy(u_v, sh_u.at[iota_v], add=True)
        plsc.subcore_barrier()

        @pl.when(sid == 0)
        def _():
            pltpu.sync_copy(sh_s, s2_out.at[c])
            pltpu.sync_copy(sh_u, u2_out.at[c])

    return sc_seg


def _pre0_body(x_ref, att_ref, lw_ref, lb_ref, o_ref):
    f32 = jnp.float32
    hi = lax.Precision.HIGHEST
    a1 = att_ref[0:1, :]
    w1 = lax.dot_general(a1, lw_ref[...], (((1,), (1,)), ((), ())),
                         preferred_element_type=f32, precision=hi)
    b1 = jnp.sum(lb_ref[...] * a1, keepdims=True)
    pre0 = lax.dot_general(w1, x_ref[...], (((1,), (1,)), ((), ())),
                           preferred_element_type=f32, precision=hi)
    o_ref[...] = pre0 + b1


def _final_body(w_ref, att_ref, b_ref, pre0_ref, s2_ref, u2_ref, o_ref):
    f32 = jnp.float32
    hi = lax.Precision.HIGHEST
    s = s2_ref[0:1, :] + s2_ref[1:2, :]
    u = u2_ref[0:1, :] + u2_ref[1:2, :]
    v = jnp.dot(u, w_ref[...], preferred_element_type=f32, precision=hi)
    a2 = att_ref[1:2, :]
    c2 = jnp.sum(v * a2, keepdims=True)
    pre = pre0_ref[...] + s * c2
    alpha = jnp.where(pre >= 0, pre, 0.2 * pre)
    t = jnp.sum(v * alpha, keepdims=True)
    o_ref[...] = jnp.maximum(s * t + b_ref[...], 0.0)


def kernel(x, edge_index, weight, bias, attention, lin_w, lin_b):
    n, in_ch = x.shape
    out_ch = weight.shape[1]
    e = edge_index.shape[1]
    src2d = edge_index[0].reshape(e // 128, 128)
    dst2d = edge_index[1].reshape(e // 128, 128)
    att2 = attention.reshape(2, out_ch)

    s2, u2 = _make_sc_seg(n, e // 128, in_ch)(src2d, dst2d, x)

    pre0 = pl.pallas_call(
        _pre0_body,
        out_shape=jax.ShapeDtypeStruct((1, n), jnp.float32),
    )(x, att2, lin_w, lin_b.reshape(1, out_ch))

    out = pl.pallas_call(
        _final_body,
        out_shape=jax.ShapeDtypeStruct((1, n), jnp.float32),
    )(weight, att2, bias.reshape(1, out_ch), pre0, s2, u2)
    return out.reshape(n)
